# baseline (speedup 1.0000x reference)
import jax
import jax.numpy as jnp
from jax.experimental import pallas as pl


AX0 = 64
AX1 = 64
DIM = 1024
SEQ = AX0 * AX1
BATCH = 4
I_BLK = 32


def _body(w0_ref, w1_ref, o_ref):
    w0b = w0_ref[...]
    w1b = w1_ref[...]
    o_ref[...] = (w0b[:, None, :] + w1b[None, :, :]).reshape(
        1, I_BLK * AX1, DIM
    )


def kernel(x, w0, w1):
    w0f = w0.reshape(AX0, DIM)
    w1f = w1.reshape(AX1, DIM)
    out = pl.pallas_call(
        _body,
        grid=(BATCH, AX0 // I_BLK),
        in_specs=[
            pl.BlockSpec((I_BLK, DIM), lambda b, i: (i, 0)),
            pl.BlockSpec((AX1, DIM), lambda b, i: (0, 0)),
        ],
        out_specs=pl.BlockSpec((1, I_BLK * AX1, DIM), lambda b, i: (b, i, 0)),
        out_shape=jax.ShapeDtypeStruct((BATCH, SEQ, DIM), x.dtype),
    )(w0f, w1f)
    return out


# speedup vs baseline: 1.0142x; 1.0142x over previous
import jax
import jax.numpy as jnp
from jax.experimental import pallas as pl


AX0 = 64
AX1 = 64
DIM = 1024
SEQ = AX0 * AX1
BATCH = 4
I_BLK = 8


def _body(w0_ref, w1_ref, o_ref):
    w0b = w0_ref[...]
    w1b = w1_ref[...]
    s = (w0b[:, None, :] + w1b[None, :, :]).reshape(I_BLK * AX1, DIM)
    o_ref[...] = jnp.broadcast_to(s[None], (BATCH, I_BLK * AX1, DIM))


def kernel(x, w0, w1):
    w0f = w0.reshape(AX0, DIM)
    w1f = w1.reshape(AX1, DIM)
    out = pl.pallas_call(
        _body,
        grid=(AX0 // I_BLK,),
        in_specs=[
            pl.BlockSpec((I_BLK, DIM), lambda i: (i, 0)),
            pl.BlockSpec((AX1, DIM), lambda i: (0, 0)),
        ],
        out_specs=pl.BlockSpec((BATCH, I_BLK * AX1, DIM), lambda i: (0, i, 0)),
        out_shape=jax.ShapeDtypeStruct((BATCH, SEQ, DIM), x.dtype),
    )(w0f, w1f)
    return out
